# Initial kernel scaffold; baseline (speedup 1.0000x reference)
import jax
import jax.numpy as jnp
from jax.experimental import pallas as pl


def kernel(x, table):
    raise NotImplementedError("write your pallas kernel here")


# speedup vs baseline: 4.1379x; 4.1379x over previous
import functools

import jax
import jax.numpy as jnp
from jax import lax
from jax.experimental import pallas as pl
from jax.experimental.pallas import tpu as pltpu
from jax.experimental.pallas import tpu_sc as plsc

_L = 128
_K = 8


def kernel(x, table):
    B, S = x.shape
    V, D = table.shape
    N = B * S

    info = plsc.get_sparse_core_info()
    NC, NS = info.num_cores, info.num_subcores
    NW = NC * NS

    rows_total = N // _L
    rows_per_w = rows_total // NW
    chunks = rows_per_w // _K

    idx2d = x.reshape(rows_total, _L).astype(jnp.int32)

    mesh = plsc.VectorSubcoreMesh(core_axis_name="c", subcore_axis_name="s")

    @functools.partial(
        pl.kernel,
        mesh=mesh,
        compiler_params=pltpu.CompilerParams(use_tc_tiling_on_sc=False),
        out_type=jax.ShapeDtypeStruct((rows_total, _L, D), jnp.float32),
        scratch_types=[
            pltpu.VMEM((_K, _L), jnp.int32),
            pltpu.VMEM((_K, _L, D), jnp.float32),
            pltpu.SemaphoreType.DMA,
        ],
    )
    def emb(idx_hbm, table_hbm, out_hbm, idx_v, rows_v, sem):
        wid = lax.axis_index("s") * NC + lax.axis_index("c")
        row_base = wid * rows_per_w

        def body(g, carry):
            rb = row_base + g * _K
            pltpu.sync_copy(idx_hbm.at[pl.ds(rb, _K)], idx_v)
            copies = [
                pltpu.async_copy(table_hbm.at[idx_v.at[j]], rows_v.at[j], sem)
                for j in range(_K)
            ]
            for c in copies:
                c.wait()
            pltpu.sync_copy(rows_v, out_hbm.at[pl.ds(rb, _K)])
            return carry

        lax.fori_loop(0, chunks, body, 0)

    out = emb(idx2d, table)
    return out.reshape(B, S, D)


# speedup vs baseline: 4.2590x; 1.0293x over previous
import functools

import jax
import jax.numpy as jnp
from jax import lax
from jax.experimental import pallas as pl
from jax.experimental.pallas import tpu as pltpu
from jax.experimental.pallas import tpu_sc as plsc

_L = 128
_K = 2
_NBUF = 4


def kernel(x, table):
    B, S = x.shape
    V, D = table.shape
    N = B * S

    info = plsc.get_sparse_core_info()
    NC, NS = info.num_cores, info.num_subcores
    NW = NC * NS

    rows_total = N // _L
    rows_per_w = rows_total // NW
    chunks = rows_per_w // _K
    steady = chunks - _NBUF
    assert rows_total % NW == 0 and rows_per_w % _K == 0
    assert steady % _NBUF == 0 and steady >= 0

    idx2d = x.reshape(rows_total, _L).astype(jnp.int32)

    mesh = plsc.VectorSubcoreMesh(core_axis_name="c", subcore_axis_name="s")

    @functools.partial(
        pl.kernel,
        mesh=mesh,
        compiler_params=pltpu.CompilerParams(use_tc_tiling_on_sc=False),
        out_type=jax.ShapeDtypeStruct((rows_total, _L, D), jnp.float32),
        scratch_types=[
            pltpu.VMEM((rows_per_w, _L), jnp.int32),
            pltpu.VMEM((_NBUF, _K, _L, D), jnp.float32),
            [pltpu.SemaphoreType.DMA] * _NBUF,
        ],
    )
    def emb(idx_hbm, table_hbm, out_hbm, idx_all, rows_v, gsems):
        wid = lax.axis_index("s") * NC + lax.axis_index("c")
        row_base = wid * rows_per_w

        pltpu.sync_copy(idx_hbm.at[pl.ds(row_base, rows_per_w)], idx_all)

        def fire(g, b):
            for j in range(_K):
                pltpu.async_copy(
                    table_hbm.at[idx_all.at[g * _K + j]],
                    rows_v.at[b, j],
                    gsems[b],
                )

        def drain(g, b):
            pltpu.make_async_copy(
                out_hbm.at[pl.ds(0, _K)], rows_v.at[b], gsems[b]
            ).wait()
            pltpu.sync_copy(
                rows_v.at[b], out_hbm.at[pl.ds(row_base + g * _K, _K)]
            )

        for b in range(_NBUF):
            fire(b, b)

        def body(i, carry):
            G = i * _NBUF
            for b in range(_NBUF):
                drain(G + b, b)
                fire(G + b + _NBUF, b)
            return carry

        lax.fori_loop(0, steady // _NBUF, body, 0)

        for b in range(_NBUF):
            drain(steady + b, b)

    out = emb(idx2d, table)
    return out.reshape(B, S, D)
